# baseline (speedup 1.0000x reference)
import functools
import math

import jax
import jax.numpy as jnp
from jax import lax
from jax.experimental import pallas as pl
from jax.experimental.pallas import tpu as pltpu
from jax.experimental.pallas import tpu_sc as plsc

KSEL = 16
QT = 256
KB = 4096
NEG = float("-inf")

SC_CORES = 2
SC_SUBCORES = 16
SC_WORKERS = SC_CORES * SC_SUBCORES
GATHER_CHUNK = 128


def _topk_body(K_total, nkb, q_ref, k_ref, vals_out, idx_out,
               sims_sc, vals_sc, idx_sc, L_sc, AR_sc, EXR_sc, go_sc):
    kb = pl.program_id(1)
    R = KB // 128

    @pl.when(kb == 0)
    def _init():
        vals_sc[...] = jnp.full((QT, KSEL), NEG, jnp.float32)
        idx_sc[...] = jnp.zeros((QT, KSEL), jnp.int32)

    q = q_ref[...]
    kblk = k_ref[...]
    sims = lax.dot_general(q, kblk, (((1,), (1,)), ((), ())),
                           preferred_element_type=jnp.float32)

    col0 = kb * KB
    lcol = lax.broadcasted_iota(jnp.int32, (QT, KB), 1)
    sims = jnp.where(col0 + lcol < K_total, sims, NEG)
    sims_sc[...] = sims

    L0 = jnp.full((QT, 128), NEG, jnp.float32)
    for r in range(R):
        L0 = jnp.maximum(L0, sims[:, r * 128:(r + 1) * 128])
    L_sc[...] = L0

    lane = lax.broadcasted_iota(jnp.int32, (QT, 128), 1)
    BIGI = jnp.int32(2**30)

    go0 = jnp.any(jnp.max(L0, axis=1, keepdims=True) > vals_sc[:, 0:1])

    def ocond(go):
        return go

    def obody(go):
        L = L_sc[...]
        AR = jnp.full((QT, 128), BIGI, jnp.int32)
        for r in range(R):
            s_r = sims_sc[:, pl.ds(r * 128, 128)]
            AR = jnp.where((s_r == L) & (AR == BIGI), r, AR)
        AR_sc[...] = AR
        EXR_sc[...] = jnp.full((QT, 128), -1, jnp.int32)
        go_sc[0] = 1

        for _ in range(KSEL):
            @pl.when(go_sc[0] == 1)
            def _step():
                L = L_sc[...]
                AR = AR_sc[...]
                tv = vals_sc[...]
                ti = idx_sc[...]
                m = jnp.max(L, axis=1, keepdims=True)
                beat = m > tv[:, 0:1]
                G = AR * 128 + lane
                g = jnp.min(jnp.where(L == m, G, BIGI), axis=1,
                            keepdims=True)
                al = jnp.bitwise_and(g, 127)
                tv_s = jnp.concatenate(
                    [tv[:, 1:], jnp.full((QT, 1), jnp.inf, jnp.float32)],
                    axis=1)
                ti_s = jnp.concatenate(
                    [ti[:, 1:], jnp.zeros((QT, 1), jnp.int32)], axis=1)
                c1 = beat & (tv_s <= m)
                c2 = beat & (tv <= m)
                vals_sc[...] = jnp.where(c1, tv_s, jnp.where(c2, m, tv))
                idx_sc[...] = jnp.where(c1, ti_s,
                                        jnp.where(c2, col0 + g, ti))
                exl = beat & (lane == al)
                EXR_sc[...] = jnp.where(exl, AR, EXR_sc[...])
                L = jnp.where(exl, NEG, L)
                L_sc[...] = L
                new_thresh = jnp.where(
                    beat, jnp.minimum(tv[:, 1:2], m), tv[:, 0:1])
                goi = jnp.any(jnp.max(L, axis=1, keepdims=True)
                              > new_thresh)
                go_sc[0] = goi.astype(jnp.int32)

        EXR = EXR_sc[...]
        L2 = jnp.full((QT, 128), NEG, jnp.float32)
        for r in range(R):
            s_r = sims_sc[:, pl.ds(r * 128, 128)]
            s_r = jnp.where(EXR == r, NEG, s_r)
            sims_sc[:, pl.ds(r * 128, 128)] = s_r
            L2 = jnp.maximum(L2, s_r)
        L_sc[...] = L2
        return jnp.any(jnp.max(L2, axis=1, keepdims=True)
                       > vals_sc[:, 0:1])

    lax.while_loop(ocond, obody, go0)

    @pl.when(kb == nkb - 1)
    def _emit():
        vals_out[...] = vals_sc[...] * jnp.float32(
            1.0 / math.sqrt(q_ref.shape[-1]))
        idx_out[...] = idx_sc[...]


def _topk_call(queries, keys):
    Q, D = queries.shape
    K_total = keys.shape[0]
    nkb = pl.cdiv(K_total, KB)
    grid = (Q // QT, nkb)
    return pl.pallas_call(
        functools.partial(_topk_body, K_total, nkb),
        grid=grid,
        in_specs=[
            pl.BlockSpec((QT, D), lambda q, kb: (q, 0)),
            pl.BlockSpec((KB, D), lambda q, kb: (kb, 0)),
        ],
        out_specs=[
            pl.BlockSpec((QT, KSEL), lambda q, kb: (q, 0)),
            pl.BlockSpec((QT, KSEL), lambda q, kb: (q, 0)),
        ],
        out_shape=[
            jax.ShapeDtypeStruct((Q, KSEL), jnp.float32),
            jax.ShapeDtypeStruct((Q, KSEL), jnp.int32),
        ],
        scratch_shapes=[
            pltpu.VMEM((QT, KB), jnp.float32),
            pltpu.VMEM((QT, KSEL), jnp.float32),
            pltpu.VMEM((QT, KSEL), jnp.int32),
            pltpu.VMEM((QT, 128), jnp.float32),
            pltpu.VMEM((QT, 128), jnp.int32),
            pltpu.VMEM((QT, 128), jnp.int32),
            pltpu.SMEM((1,), jnp.int32),
        ],
        compiler_params=pltpu.CompilerParams(
            dimension_semantics=("arbitrary", "arbitrary")),
    )(queries, keys)


def _gather_call(values, idx_flat):
    B = idx_flat.shape[0]
    D = values.shape[1]
    bpw = B // SC_WORKERS
    nchunk = bpw // GATHER_CHUNK
    idx3 = idx_flat.reshape(SC_WORKERS, nchunk, GATHER_CHUNK)
    mesh = plsc.VectorSubcoreMesh(core_axis_name="c", subcore_axis_name="s")

    @functools.partial(
        pl.kernel,
        mesh=mesh,
        out_type=jax.ShapeDtypeStruct((B, D), jnp.float32),
        scratch_types=[
            pltpu.VMEM((nchunk, GATHER_CHUNK), jnp.int32),
            pltpu.VMEM((bpw, D), jnp.float32),
            pltpu.SemaphoreType.DMA,
        ],
    )
    def gather(values_hbm, idx_hbm, out_hbm, idx_v, rows_v, sem):
        wid = lax.axis_index("s") * SC_CORES + lax.axis_index("c")
        pltpu.sync_copy(idx_hbm.at[wid], idx_v)
        copies = [
            pltpu.async_copy(
                values_hbm.at[idx_v.at[j]],
                rows_v.at[pl.ds(j * GATHER_CHUNK, GATHER_CHUNK)],
                sem,
            )
            for j in range(nchunk)
        ]
        for cp in copies:
            cp.wait()
        pltpu.sync_copy(rows_v, out_hbm.at[pl.ds(wid * bpw, bpw)])

    return gather(values, idx3)


def _finish_body(vals_ref, g_ref, w_ref, o_ref):
    tv = vals_ref[...]
    m = jnp.max(tv, axis=1, keepdims=True)
    e = jnp.exp(tv - m)
    w = e / jnp.sum(e, axis=1, keepdims=True)
    g = g_ref[...].reshape(QT, KSEL, g_ref.shape[-1])
    agg = jnp.sum(w[..., None] * g, axis=1)
    o_ref[...] = jnp.dot(agg, w_ref[...], preferred_element_type=jnp.float32)


def _finish_call(top_vals, gathered, W):
    Q = top_vals.shape[0]
    D = W.shape[0]
    grid = (Q // QT,)
    return pl.pallas_call(
        _finish_body,
        grid=grid,
        in_specs=[
            pl.BlockSpec((QT, KSEL), lambda q: (q, 0)),
            pl.BlockSpec((QT * KSEL, D), lambda q: (q, 0)),
            pl.BlockSpec((D, D), lambda q: (0, 0)),
        ],
        out_specs=pl.BlockSpec((QT, D), lambda q: (q, 0)),
        out_shape=jax.ShapeDtypeStruct((Q, D), jnp.float32),
    )(top_vals, gathered, W)


def kernel(queries, keys, values, W, k):
    del k
    top_vals, top_idx = _topk_call(queries, keys)
    gathered = _gather_call(values, top_idx.reshape(-1))
    return _finish_call(top_vals, gathered, W)


# speedup vs baseline: 1.4521x; 1.4521x over previous
import functools
import math

import jax
import jax.numpy as jnp
from jax import lax
from jax.experimental import pallas as pl
from jax.experimental.pallas import tpu as pltpu
from jax.experimental.pallas import tpu_sc as plsc

KSEL = 16
QT = 256
KB = 8192
NEG = float("-inf")

SC_CORES = 2
SC_SUBCORES = 16
SC_WORKERS = SC_CORES * SC_SUBCORES
GATHER_CHUNK = 128


def _topk_body(K_total, nkb, q_ref, k_ref, vals_out, idx_out,
               sims_sc, vals_sc, idx_sc, L_sc, AR_sc, EXR_sc, go_sc):
    kb = pl.program_id(1)
    R = KB // 128

    @pl.when(kb == 0)
    def _init():
        vals_sc[...] = jnp.full((QT, KSEL), NEG, jnp.float32)
        idx_sc[...] = jnp.zeros((QT, KSEL), jnp.int32)

    q = q_ref[...]
    kblk = k_ref[...]
    sims = lax.dot_general(q, kblk, (((1,), (1,)), ((), ())),
                           preferred_element_type=jnp.float32)

    col0 = kb * KB
    lcol = lax.broadcasted_iota(jnp.int32, (QT, KB), 1)
    sims = jnp.where(col0 + lcol < K_total, sims, NEG)
    sims_sc[...] = sims

    L0 = jnp.full((QT, 128), NEG, jnp.float32)
    for r in range(R):
        L0 = jnp.maximum(L0, sims[:, r * 128:(r + 1) * 128])
    L_sc[...] = L0

    lane = lax.broadcasted_iota(jnp.int32, (QT, 128), 1)
    i16 = lax.broadcasted_iota(jnp.int32, (QT, KSEL), 1)
    BIGI = jnp.int32(2**30)

    go0 = jnp.any(jnp.max(L0, axis=1, keepdims=True)
                  > jnp.min(vals_sc[...], axis=1, keepdims=True))

    def ocond(go):
        return go

    def obody(go):
        L = L_sc[...]
        AR = jnp.full((QT, 128), BIGI, jnp.int32)
        for r in range(R):
            s_r = sims_sc[:, pl.ds(r * 128, 128)]
            AR = jnp.where((s_r == L) & (AR == BIGI), r, AR)
        AR_sc[...] = AR
        EXR_sc[...] = jnp.full((QT, 128), -1, jnp.int32)
        go_sc[0] = 1

        for _ in range(KSEL):
            @pl.when(go_sc[0] == 1)
            def _step():
                L = L_sc[...]
                AR = AR_sc[...]
                tv = vals_sc[...]
                thresh = jnp.min(tv, axis=1, keepdims=True)
                m = jnp.max(L, axis=1, keepdims=True)
                beat = m > thresh
                G = AR * 128 + lane
                g = jnp.min(jnp.where(L == m, G, BIGI), axis=1,
                            keepdims=True)
                al = jnp.bitwise_and(g, 127)
                pos = jnp.min(jnp.where(tv == thresh, i16, KSEL), axis=1,
                              keepdims=True)
                upd = beat & (i16 == pos)
                new_tv = jnp.where(upd, m, tv)
                vals_sc[...] = new_tv
                idx_sc[...] = jnp.where(upd, col0 + g, idx_sc[...])
                exl = beat & (lane == al)
                EXR_sc[...] = jnp.where(exl, AR, EXR_sc[...])
                L = jnp.where(exl, NEG, L)
                L_sc[...] = L
                goi = jnp.any(jnp.max(L, axis=1, keepdims=True)
                              > jnp.min(new_tv, axis=1, keepdims=True))
                go_sc[0] = goi.astype(jnp.int32)

        EXR = EXR_sc[...]
        L2 = jnp.full((QT, 128), NEG, jnp.float32)
        for r in range(R):
            s_r = sims_sc[:, pl.ds(r * 128, 128)]
            s_r = jnp.where(EXR == r, NEG, s_r)
            sims_sc[:, pl.ds(r * 128, 128)] = s_r
            L2 = jnp.maximum(L2, s_r)
        L_sc[...] = L2
        return jnp.any(jnp.max(L2, axis=1, keepdims=True)
                       > jnp.min(vals_sc[...], axis=1, keepdims=True))

    lax.while_loop(ocond, obody, go0)

    @pl.when(kb == nkb - 1)
    def _emit():
        vals_out[...] = vals_sc[...] * jnp.float32(
            1.0 / math.sqrt(q_ref.shape[-1]))
        idx_out[...] = idx_sc[...]


def _topk_call(queries, keys):
    Q, D = queries.shape
    K_total = keys.shape[0]
    nkb = pl.cdiv(K_total, KB)
    grid = (Q // QT, nkb)
    return pl.pallas_call(
        functools.partial(_topk_body, K_total, nkb),
        grid=grid,
        in_specs=[
            pl.BlockSpec((QT, D), lambda q, kb: (q, 0)),
            pl.BlockSpec((KB, D), lambda q, kb: (kb, 0)),
        ],
        out_specs=[
            pl.BlockSpec((QT, KSEL), lambda q, kb: (q, 0)),
            pl.BlockSpec((QT, KSEL), lambda q, kb: (q, 0)),
        ],
        out_shape=[
            jax.ShapeDtypeStruct((Q, KSEL), jnp.float32),
            jax.ShapeDtypeStruct((Q, KSEL), jnp.int32),
        ],
        scratch_shapes=[
            pltpu.VMEM((QT, KB), jnp.float32),
            pltpu.VMEM((QT, KSEL), jnp.float32),
            pltpu.VMEM((QT, KSEL), jnp.int32),
            pltpu.VMEM((QT, 128), jnp.float32),
            pltpu.VMEM((QT, 128), jnp.int32),
            pltpu.VMEM((QT, 128), jnp.int32),
            pltpu.SMEM((1,), jnp.int32),
        ],
        compiler_params=pltpu.CompilerParams(
            dimension_semantics=("arbitrary", "arbitrary")),
    )(queries, keys)


def _gather_call(values, idx_flat):
    B = idx_flat.shape[0]
    D = values.shape[1]
    bpw = B // SC_WORKERS
    nchunk = bpw // GATHER_CHUNK
    idx3 = idx_flat.reshape(SC_WORKERS, nchunk, GATHER_CHUNK)
    mesh = plsc.VectorSubcoreMesh(core_axis_name="c", subcore_axis_name="s")

    @functools.partial(
        pl.kernel,
        mesh=mesh,
        out_type=jax.ShapeDtypeStruct((B, D), jnp.float32),
        scratch_types=[
            pltpu.VMEM((nchunk, GATHER_CHUNK), jnp.int32),
            pltpu.VMEM((bpw, D), jnp.float32),
            pltpu.SemaphoreType.DMA,
        ],
    )
    def gather(values_hbm, idx_hbm, out_hbm, idx_v, rows_v, sem):
        wid = lax.axis_index("s") * SC_CORES + lax.axis_index("c")
        pltpu.sync_copy(idx_hbm.at[wid], idx_v)
        copies = [
            pltpu.async_copy(
                values_hbm.at[idx_v.at[j]],
                rows_v.at[pl.ds(j * GATHER_CHUNK, GATHER_CHUNK)],
                sem,
            )
            for j in range(nchunk)
        ]
        for cp in copies:
            cp.wait()
        pltpu.sync_copy(rows_v, out_hbm.at[pl.ds(wid * bpw, bpw)])

    return gather(values, idx3)


def _finish_body(vals_ref, g_ref, w_ref, o_ref):
    tv = vals_ref[...]
    m = jnp.max(tv, axis=1, keepdims=True)
    e = jnp.exp(tv - m)
    w = e / jnp.sum(e, axis=1, keepdims=True)
    g = g_ref[...].reshape(QT, KSEL, g_ref.shape[-1])
    agg = jnp.sum(w[..., None] * g, axis=1)
    o_ref[...] = jnp.dot(agg, w_ref[...], preferred_element_type=jnp.float32)


def _finish_call(top_vals, gathered, W):
    Q = top_vals.shape[0]
    D = W.shape[0]
    grid = (Q // QT,)
    return pl.pallas_call(
        _finish_body,
        grid=grid,
        in_specs=[
            pl.BlockSpec((QT, KSEL), lambda q: (q, 0)),
            pl.BlockSpec((QT * KSEL, D), lambda q: (q, 0)),
            pl.BlockSpec((D, D), lambda q: (0, 0)),
        ],
        out_specs=pl.BlockSpec((QT, D), lambda q: (q, 0)),
        out_shape=jax.ShapeDtypeStruct((Q, D), jnp.float32),
    )(top_vals, gathered, W)


def kernel(queries, keys, values, W, k):
    del k
    top_vals, top_idx = _topk_call(queries, keys)
    gathered = _gather_call(values, top_idx.reshape(-1))
    return _finish_call(top_vals, gathered, W)


# speedup vs baseline: 1.6453x; 1.1331x over previous
import functools
import math

import jax
import jax.numpy as jnp
from jax import lax
from jax.experimental import pallas as pl
from jax.experimental.pallas import tpu as pltpu
from jax.experimental.pallas import tpu_sc as plsc

KSEL = 16
QT = 256
KB = 16384
NEG = float("-inf")

SC_CORES = 2
SC_SUBCORES = 16
SC_WORKERS = SC_CORES * SC_SUBCORES
GATHER_CHUNK = 128


def _topk_body(K_total, nkb, q_ref, k_ref, vals_out, idx_out,
               sims_sc, vals_sc, idx_sc, L_sc, AR_sc, EXR_sc, go_sc):
    kb = pl.program_id(1)
    R = KB // 128

    @pl.when(kb == 0)
    def _init():
        vals_sc[...] = jnp.full((QT, KSEL), NEG, jnp.float32)
        idx_sc[...] = jnp.zeros((QT, KSEL), jnp.int32)

    q = q_ref[...]
    kblk = k_ref[...]
    sims = lax.dot_general(q, kblk, (((1,), (1,)), ((), ())),
                           preferred_element_type=jnp.float32)

    col0 = kb * KB
    lcol = lax.broadcasted_iota(jnp.int32, (QT, KB), 1)
    sims = jnp.where(col0 + lcol < K_total, sims, NEG)
    sims_sc[...] = sims

    L0 = jnp.full((QT, 128), NEG, jnp.float32)
    for r in range(R):
        L0 = jnp.maximum(L0, sims[:, r * 128:(r + 1) * 128])
    L_sc[...] = L0

    lane = lax.broadcasted_iota(jnp.int32, (QT, 128), 1)
    i16 = lax.broadcasted_iota(jnp.int32, (QT, KSEL), 1)
    BIGI = jnp.int32(2**30)

    go0 = jnp.any(jnp.max(L0, axis=1, keepdims=True)
                  > jnp.min(vals_sc[...], axis=1, keepdims=True))

    def ocond(go):
        return go

    def obody(go):
        L = L_sc[...]
        AR = jnp.full((QT, 128), BIGI, jnp.int32)
        for r in range(R):
            s_r = sims_sc[:, pl.ds(r * 128, 128)]
            AR = jnp.where((s_r == L) & (AR == BIGI), r, AR)
        AR_sc[...] = AR
        EXR_sc[...] = jnp.full((QT, 128), -1, jnp.int32)
        go_sc[0] = 1

        for _ in range(KSEL):
            @pl.when(go_sc[0] == 1)
            def _step():
                L = L_sc[...]
                AR = AR_sc[...]
                tv = vals_sc[...]
                thresh = jnp.min(tv, axis=1, keepdims=True)
                m = jnp.max(L, axis=1, keepdims=True)
                beat = m > thresh
                G = AR * 128 + lane
                g = jnp.min(jnp.where(L == m, G, BIGI), axis=1,
                            keepdims=True)
                al = jnp.bitwise_and(g, 127)
                pos = jnp.min(jnp.where(tv == thresh, i16, KSEL), axis=1,
                              keepdims=True)
                upd = beat & (i16 == pos)
                new_tv = jnp.where(upd, m, tv)
                vals_sc[...] = new_tv
                idx_sc[...] = jnp.where(upd, col0 + g, idx_sc[...])
                exl = beat & (lane == al)
                EXR_sc[...] = jnp.where(exl, AR, EXR_sc[...])
                L = jnp.where(exl, NEG, L)
                L_sc[...] = L
                goi = jnp.any(jnp.max(L, axis=1, keepdims=True)
                              > jnp.min(new_tv, axis=1, keepdims=True))
                go_sc[0] = goi.astype(jnp.int32)

        EXR = EXR_sc[...]
        L2 = jnp.full((QT, 128), NEG, jnp.float32)
        for r in range(R):
            s_r = sims_sc[:, pl.ds(r * 128, 128)]
            s_r = jnp.where(EXR == r, NEG, s_r)
            sims_sc[:, pl.ds(r * 128, 128)] = s_r
            L2 = jnp.maximum(L2, s_r)
        L_sc[...] = L2
        return jnp.any(jnp.max(L2, axis=1, keepdims=True)
                       > jnp.min(vals_sc[...], axis=1, keepdims=True))

    lax.while_loop(ocond, obody, go0)

    @pl.when(kb == nkb - 1)
    def _emit():
        vals_out[...] = vals_sc[...] * jnp.float32(
            1.0 / math.sqrt(q_ref.shape[-1]))
        idx_out[...] = idx_sc[...]


def _topk_call(queries, keys):
    Q, D = queries.shape
    K_total = keys.shape[0]
    nkb = pl.cdiv(K_total, KB)
    grid = (Q // QT, nkb)
    return pl.pallas_call(
        functools.partial(_topk_body, K_total, nkb),
        grid=grid,
        in_specs=[
            pl.BlockSpec((QT, D), lambda q, kb: (q, 0)),
            pl.BlockSpec((KB, D), lambda q, kb: (kb, 0)),
        ],
        out_specs=[
            pl.BlockSpec((QT, KSEL), lambda q, kb: (q, 0)),
            pl.BlockSpec((QT, KSEL), lambda q, kb: (q, 0)),
        ],
        out_shape=[
            jax.ShapeDtypeStruct((Q, KSEL), jnp.float32),
            jax.ShapeDtypeStruct((Q, KSEL), jnp.int32),
        ],
        scratch_shapes=[
            pltpu.VMEM((QT, KB), jnp.float32),
            pltpu.VMEM((QT, KSEL), jnp.float32),
            pltpu.VMEM((QT, KSEL), jnp.int32),
            pltpu.VMEM((QT, 128), jnp.float32),
            pltpu.VMEM((QT, 128), jnp.int32),
            pltpu.VMEM((QT, 128), jnp.int32),
            pltpu.SMEM((1,), jnp.int32),
        ],
        compiler_params=pltpu.CompilerParams(
            dimension_semantics=("arbitrary", "arbitrary")),
    )(queries, keys)


def _gather_call(values, idx_flat):
    B = idx_flat.shape[0]
    D = values.shape[1]
    bpw = B // SC_WORKERS
    nchunk = bpw // GATHER_CHUNK
    idx3 = idx_flat.reshape(SC_WORKERS, nchunk, GATHER_CHUNK)
    mesh = plsc.VectorSubcoreMesh(core_axis_name="c", subcore_axis_name="s")

    @functools.partial(
        pl.kernel,
        mesh=mesh,
        out_type=jax.ShapeDtypeStruct((B, D), jnp.float32),
        scratch_types=[
            pltpu.VMEM((nchunk, GATHER_CHUNK), jnp.int32),
            pltpu.VMEM((bpw, D), jnp.float32),
            pltpu.SemaphoreType.DMA,
        ],
    )
    def gather(values_hbm, idx_hbm, out_hbm, idx_v, rows_v, sem):
        wid = lax.axis_index("s") * SC_CORES + lax.axis_index("c")
        pltpu.sync_copy(idx_hbm.at[wid], idx_v)
        copies = [
            pltpu.async_copy(
                values_hbm.at[idx_v.at[j]],
                rows_v.at[pl.ds(j * GATHER_CHUNK, GATHER_CHUNK)],
                sem,
            )
            for j in range(nchunk)
        ]
        for cp in copies:
            cp.wait()
        pltpu.sync_copy(rows_v, out_hbm.at[pl.ds(wid * bpw, bpw)])

    return gather(values, idx3)


def _finish_body(vals_ref, g_ref, w_ref, o_ref):
    tv = vals_ref[...]
    m = jnp.max(tv, axis=1, keepdims=True)
    e = jnp.exp(tv - m)
    w = e / jnp.sum(e, axis=1, keepdims=True)
    g = g_ref[...].reshape(QT, KSEL, g_ref.shape[-1])
    agg = jnp.sum(w[..., None] * g, axis=1)
    o_ref[...] = jnp.dot(agg, w_ref[...], preferred_element_type=jnp.float32)


def _finish_call(top_vals, gathered, W):
    Q = top_vals.shape[0]
    D = W.shape[0]
    grid = (Q // QT,)
    return pl.pallas_call(
        _finish_body,
        grid=grid,
        in_specs=[
            pl.BlockSpec((QT, KSEL), lambda q: (q, 0)),
            pl.BlockSpec((QT * KSEL, D), lambda q: (q, 0)),
            pl.BlockSpec((D, D), lambda q: (0, 0)),
        ],
        out_specs=pl.BlockSpec((QT, D), lambda q: (q, 0)),
        out_shape=jax.ShapeDtypeStruct((Q, D), jnp.float32),
    )(top_vals, gathered, W)


def kernel(queries, keys, values, W, k):
    del k
    top_vals, top_idx = _topk_call(queries, keys)
    gathered = _gather_call(values, top_idx.reshape(-1))
    return _finish_call(top_vals, gathered, W)


# speedup vs baseline: 1.8223x; 1.1076x over previous
import functools
import math

import jax
import jax.numpy as jnp
from jax import lax
from jax.experimental import pallas as pl
from jax.experimental.pallas import tpu as pltpu
from jax.experimental.pallas import tpu_sc as plsc

KSEL = 16
QT = 256
KB = 25088
NEG = float("-inf")

SC_CORES = 2
SC_SUBCORES = 16
SC_WORKERS = SC_CORES * SC_SUBCORES
GATHER_CHUNK = 128


def _topk_body(K_total, nkb, q_ref, k_ref, vals_out, idx_out,
               sims_sc, vals_sc, idx_sc, L_sc, AR_sc, EXR_sc, go_sc):
    kb = pl.program_id(1)
    R = KB // 128

    @pl.when(kb == 0)
    def _init():
        vals_sc[...] = jnp.full((QT, KSEL), NEG, jnp.float32)
        idx_sc[...] = jnp.zeros((QT, KSEL), jnp.int32)

    q = q_ref[...]
    kblk = k_ref[...]
    sims = lax.dot_general(q, kblk, (((1,), (1,)), ((), ())),
                           preferred_element_type=jnp.float32)

    col0 = kb * KB
    lcol = lax.broadcasted_iota(jnp.int32, (QT, KB), 1)
    sims = jnp.where(col0 + lcol < K_total, sims, NEG)
    sims_sc[...] = sims

    L0 = jnp.full((QT, 128), NEG, jnp.float32)
    for r in range(R):
        L0 = jnp.maximum(L0, sims[:, r * 128:(r + 1) * 128])
    L_sc[...] = L0

    lane = lax.broadcasted_iota(jnp.int32, (QT, 128), 1)
    i16 = lax.broadcasted_iota(jnp.int32, (QT, KSEL), 1)
    BIGI = jnp.int32(2**30)

    go0 = jnp.any(jnp.max(L0, axis=1, keepdims=True)
                  > jnp.min(vals_sc[...], axis=1, keepdims=True))

    def ocond(go):
        return go

    def obody(go):
        L = L_sc[...]
        AR = jnp.full((QT, 128), BIGI, jnp.int32)
        for r in range(R):
            s_r = sims_sc[:, pl.ds(r * 128, 128)]
            AR = jnp.where((s_r == L) & (AR == BIGI), r, AR)
        AR_sc[...] = AR
        EXR_sc[...] = jnp.full((QT, 128), -1, jnp.int32)
        go_sc[0] = 1

        for _ in range(KSEL):
            @pl.when(go_sc[0] == 1)
            def _step():
                L = L_sc[...]
                AR = AR_sc[...]
                tv = vals_sc[...]
                thresh = jnp.min(tv, axis=1, keepdims=True)
                m = jnp.max(L, axis=1, keepdims=True)
                beat = m > thresh
                G = AR * 128 + lane
                g = jnp.min(jnp.where(L == m, G, BIGI), axis=1,
                            keepdims=True)
                al = jnp.bitwise_and(g, 127)
                pos = jnp.min(jnp.where(tv == thresh, i16, KSEL), axis=1,
                              keepdims=True)
                upd = beat & (i16 == pos)
                new_tv = jnp.where(upd, m, tv)
                vals_sc[...] = new_tv
                idx_sc[...] = jnp.where(upd, col0 + g, idx_sc[...])
                exl = beat & (lane == al)
                EXR_sc[...] = jnp.where(exl, AR, EXR_sc[...])
                L = jnp.where(exl, NEG, L)
                L_sc[...] = L
                goi = jnp.any(jnp.max(L, axis=1, keepdims=True)
                              > jnp.min(new_tv, axis=1, keepdims=True))
                go_sc[0] = goi.astype(jnp.int32)

        EXR = EXR_sc[...]
        L2 = jnp.full((QT, 128), NEG, jnp.float32)
        for r in range(R):
            s_r = sims_sc[:, pl.ds(r * 128, 128)]
            s_r = jnp.where(EXR == r, NEG, s_r)
            sims_sc[:, pl.ds(r * 128, 128)] = s_r
            L2 = jnp.maximum(L2, s_r)
        L_sc[...] = L2
        return jnp.any(jnp.max(L2, axis=1, keepdims=True)
                       > jnp.min(vals_sc[...], axis=1, keepdims=True))

    lax.while_loop(ocond, obody, go0)

    @pl.when(kb == nkb - 1)
    def _emit():
        vals_out[...] = vals_sc[...] * jnp.float32(
            1.0 / math.sqrt(q_ref.shape[-1]))
        idx_out[...] = idx_sc[...]


def _topk_call(queries, keys):
    Q, D = queries.shape
    K_total = keys.shape[0]
    nkb = pl.cdiv(K_total, KB)
    grid = (Q // QT, nkb)
    return pl.pallas_call(
        functools.partial(_topk_body, K_total, nkb),
        grid=grid,
        in_specs=[
            pl.BlockSpec((QT, D), lambda q, kb: (q, 0)),
            pl.BlockSpec((KB, D), lambda q, kb: (kb, 0)),
        ],
        out_specs=[
            pl.BlockSpec((QT, KSEL), lambda q, kb: (q, 0)),
            pl.BlockSpec((QT, KSEL), lambda q, kb: (q, 0)),
        ],
        out_shape=[
            jax.ShapeDtypeStruct((Q, KSEL), jnp.float32),
            jax.ShapeDtypeStruct((Q, KSEL), jnp.int32),
        ],
        scratch_shapes=[
            pltpu.VMEM((QT, KB), jnp.float32),
            pltpu.VMEM((QT, KSEL), jnp.float32),
            pltpu.VMEM((QT, KSEL), jnp.int32),
            pltpu.VMEM((QT, 128), jnp.float32),
            pltpu.VMEM((QT, 128), jnp.int32),
            pltpu.VMEM((QT, 128), jnp.int32),
            pltpu.SMEM((1,), jnp.int32),
        ],
        compiler_params=pltpu.CompilerParams(
            dimension_semantics=("arbitrary", "arbitrary")),
    )(queries, keys)


def _gather_call(values, idx_flat):
    B = idx_flat.shape[0]
    D = values.shape[1]
    bpw = B // SC_WORKERS
    nchunk = bpw // GATHER_CHUNK
    idx3 = idx_flat.reshape(SC_WORKERS, nchunk, GATHER_CHUNK)
    mesh = plsc.VectorSubcoreMesh(core_axis_name="c", subcore_axis_name="s")

    @functools.partial(
        pl.kernel,
        mesh=mesh,
        out_type=jax.ShapeDtypeStruct((B, D), jnp.float32),
        scratch_types=[
            pltpu.VMEM((nchunk, GATHER_CHUNK), jnp.int32),
            pltpu.VMEM((bpw, D), jnp.float32),
            pltpu.SemaphoreType.DMA,
        ],
    )
    def gather(values_hbm, idx_hbm, out_hbm, idx_v, rows_v, sem):
        wid = lax.axis_index("s") * SC_CORES + lax.axis_index("c")
        pltpu.sync_copy(idx_hbm.at[wid], idx_v)
        copies = [
            pltpu.async_copy(
                values_hbm.at[idx_v.at[j]],
                rows_v.at[pl.ds(j * GATHER_CHUNK, GATHER_CHUNK)],
                sem,
            )
            for j in range(nchunk)
        ]
        for cp in copies:
            cp.wait()
        pltpu.sync_copy(rows_v, out_hbm.at[pl.ds(wid * bpw, bpw)])

    return gather(values, idx3)


def _finish_body(vals_ref, g_ref, w_ref, o_ref):
    tv = vals_ref[...]
    m = jnp.max(tv, axis=1, keepdims=True)
    e = jnp.exp(tv - m)
    w = e / jnp.sum(e, axis=1, keepdims=True)
    g = g_ref[...].reshape(QT, KSEL, g_ref.shape[-1])
    agg = jnp.sum(w[..., None] * g, axis=1)
    o_ref[...] = jnp.dot(agg, w_ref[...], preferred_element_type=jnp.float32)


def _finish_call(top_vals, gathered, W):
    Q = top_vals.shape[0]
    D = W.shape[0]
    grid = (Q // QT,)
    return pl.pallas_call(
        _finish_body,
        grid=grid,
        in_specs=[
            pl.BlockSpec((QT, KSEL), lambda q: (q, 0)),
            pl.BlockSpec((QT * KSEL, D), lambda q: (q, 0)),
            pl.BlockSpec((D, D), lambda q: (0, 0)),
        ],
        out_specs=pl.BlockSpec((QT, D), lambda q: (q, 0)),
        out_shape=jax.ShapeDtypeStruct((Q, D), jnp.float32),
    )(top_vals, gathered, W)


def kernel(queries, keys, values, W, k):
    del k
    top_vals, top_idx = _topk_call(queries, keys)
    gathered = _gather_call(values, top_idx.reshape(-1))
    return _finish_call(top_vals, gathered, W)
